# baseline (speedup 1.0000x reference)
import functools

import jax
import jax.numpy as jnp
from jax import lax
from jax.experimental import pallas as pl
from jax.experimental.pallas import tpu as pltpu
from jax.experimental.pallas import tpu_sc as plsc

_N_ATOMS = 32768
_N_MOL = 16
_NODE_DIM = 128
_HIDDEN_DIM = 64
_N_ELEM = 119
_TILE = 8192
_GRID = _N_ATOMS // _TILE

_NC = 2
_NS = 16
_NW = _NC * _NS
_CHUNK = _N_ATOMS // _NW
_LANES = 16


def _mass_side_kernel(px_hbm, py_hbm, pz_hbm, batch_hbm, atno_hbm, mt_hbm,
                      out_hbm, px_v, py_v, pz_v, batch_v, atno_v, m_v,
                      acc_v, sem):
    cid = lax.axis_index("c")
    sid = lax.axis_index("s")
    wid = cid * _NS + sid
    base = wid * _CHUNK

    pltpu.sync_copy(atno_hbm.at[pl.ds(base, _CHUNK)], atno_v)
    pltpu.sync_copy(batch_hbm.at[pl.ds(base, _CHUNK)], batch_v)
    pltpu.sync_copy(px_hbm.at[pl.ds(base, _CHUNK)], px_v)
    pltpu.sync_copy(py_hbm.at[pl.ds(base, _CHUNK)], py_v)
    pltpu.sync_copy(pz_hbm.at[pl.ds(base, _CHUNK)], pz_v)

    for j in range(_CHUNK // 128):
        sl = pl.ds(j * 128, 128)
        pltpu.async_copy(mt_hbm.at[atno_v.at[sl]], m_v.at[sl], sem).wait()

    for mol in range(_N_MOL):
        def vec_body(j, carry, mol=mol):
            ax, ay, az, am = carry
            sl = pl.ds(j * _LANES, _LANES)
            keep = batch_v[sl] == mol
            m = jnp.where(keep, m_v[sl], 0.0)
            ax = ax + m * px_v[sl]
            ay = ay + m * py_v[sl]
            az = az + m * pz_v[sl]
            am = am + m
            return ax, ay, az, am

        z = jnp.zeros((_LANES,), jnp.float32)
        ax, ay, az, am = lax.fori_loop(0, _CHUNK // _LANES, vec_body,
                                       (z, z, z, z))
        acc_v[0, mol, :] = ax
        acc_v[1, mol, :] = ay
        acc_v[2, mol, :] = az
        acc_v[3, mol, :] = am

    pltpu.sync_copy(acc_v, out_hbm.at[wid])


def _mass_side(px, py, pz, batch1d, atno1d, mt1d):
    mesh = plsc.VectorSubcoreMesh(core_axis_name="c", subcore_axis_name="s")
    k = functools.partial(
        pl.kernel,
        out_type=jax.ShapeDtypeStruct((_NW, 4, _N_MOL, _LANES),
                                      jnp.float32),
        mesh=mesh,
        scratch_types=[
            pltpu.VMEM((_CHUNK,), jnp.float32),
            pltpu.VMEM((_CHUNK,), jnp.float32),
            pltpu.VMEM((_CHUNK,), jnp.float32),
            pltpu.VMEM((_CHUNK,), jnp.int32),
            pltpu.VMEM((_CHUNK,), jnp.int32),
            pltpu.VMEM((_CHUNK,), jnp.float32),
            pltpu.VMEM((4, _N_MOL, _LANES), jnp.float32),
            pltpu.SemaphoreType.DMA,
        ],
    )(_mass_side_kernel)
    return k(px, py, pz, batch1d, atno1d, mt1d)


def _s_side_kernel(x_ref, posT_ref, batch_ref, W1_ref, b1_ref, W2_ref,
                   b2_ref, out_ref):
    i = pl.program_id(0)

    @pl.when(i == 0)
    def _init():
        out_ref[...] = jnp.zeros_like(out_ref)

    x = x_ref[...]
    pT = posT_ref[...]
    b = batch_ref[...]

    px = pT[0:1, :]
    py = pT[1:2, :]
    pz = pT[2:3, :]
    r2 = px * px + py * py + pz * pz

    seg = jnp.where(
        lax.broadcasted_iota(jnp.int32, (_N_MOL, _TILE), 0) == b,
        1.0, 0.0)

    h = jnp.dot(x, W1_ref[...], preferred_element_type=jnp.float32)
    h = h + b1_ref[...]
    h = h * jax.nn.sigmoid(h)
    s = jnp.dot(h, W2_ref[...], preferred_element_type=jnp.float32)
    s = s + b2_ref[...]

    SF = jnp.concatenate([seg * r2, seg * px, seg * py, seg * pz, seg],
                         axis=0)
    out_ref[...] += lax.dot_general(
        SF, s, (((1,), (0,)), ((), ())),
        preferred_element_type=jnp.float32)


def _s_side(x_scalar, posT, batch2, W1, b1r, W2, b2r):
    return pl.pallas_call(
        _s_side_kernel,
        grid=(_GRID,),
        in_specs=[
            pl.BlockSpec((_TILE, _NODE_DIM), lambda i: (i, 0)),
            pl.BlockSpec((3, _TILE), lambda i: (0, i)),
            pl.BlockSpec((1, _TILE), lambda i: (0, i)),
            pl.BlockSpec((_NODE_DIM, _HIDDEN_DIM), lambda i: (0, 0)),
            pl.BlockSpec((1, _HIDDEN_DIM), lambda i: (0, 0)),
            pl.BlockSpec((_HIDDEN_DIM, 1), lambda i: (0, 0)),
            pl.BlockSpec((1, 1), lambda i: (0, 0)),
        ],
        out_specs=pl.BlockSpec((80, 1), lambda i: (0, 0)),
        out_shape=jax.ShapeDtypeStruct((80, 1), jnp.float32),
        compiler_params=pltpu.CompilerParams(
            dimension_semantics=("arbitrary",)),
    )(x_scalar, posT, batch2, W1, b1r, W2, b2r)


def kernel(x_scalar, x_spherical, pos, batch, at_no, masses_table, W1, b1,
           W2, b2):
    del x_spherical
    posT = pos.T
    batch1d = batch.astype(jnp.int32)
    batch2 = batch1d.reshape(1, _N_ATOMS)
    atno1d = at_no.astype(jnp.int32)
    mt1d = jnp.zeros((128,), jnp.float32).at[:_N_ELEM].set(masses_table)
    b1r = b1.reshape(1, _HIDDEN_DIM)
    b2r = b2.reshape(1, 1)

    sc_part = _mass_side(posT[0], posT[1], posT[2], batch1d, atno1d,
                         mt1d)
    accs = _s_side(x_scalar, posT, batch2, W1, b1r, W2, b2r)

    mass = jnp.sum(sc_part, axis=(0, 3))
    A = accs[0:16, 0]
    Bx = accs[16:32, 0]
    By = accs[32:48, 0]
    Bz = accs[48:64, 0]
    C = accs[64:80, 0]
    S = mass[3]
    den = jnp.where(S > 0.0, S, 1.0)
    cx = mass[0] / den
    cy = mass[1] / den
    cz = mass[2] / den
    res = (A - 2.0 * (Bx * cx + By * cy + Bz * cz)
           + C * (cx * cx + cy * cy + cz * cz))
    return res.reshape(_N_MOL, 1)


# speedup vs baseline: 1.0173x; 1.0173x over previous
import functools

import jax
import jax.numpy as jnp
from jax import lax
from jax.experimental import pallas as pl
from jax.experimental.pallas import tpu as pltpu
from jax.experimental.pallas import tpu_sc as plsc

_N_ATOMS = 32768
_N_MOL = 16
_NODE_DIM = 128
_HIDDEN_DIM = 64
_N_ELEM = 119
_TILE = 8192
_GRID = _N_ATOMS // _TILE

_NC = 2
_NS = 16
_NW = _NC * _NS
_CHUNK = _N_ATOMS // _NW
_LANES = 16


def _mass_side_kernel(px_hbm, py_hbm, pz_hbm, batch_hbm, atno_hbm, mt_hbm,
                      out_hbm, px_v, py_v, pz_v, batch_v, atno_v, m_v,
                      acc_v, sem):
    cid = lax.axis_index("c")
    sid = lax.axis_index("s")
    wid = cid * _NS + sid
    base = wid * _CHUNK

    pltpu.sync_copy(atno_hbm.at[pl.ds(base, _CHUNK)], atno_v)
    pltpu.sync_copy(batch_hbm.at[pl.ds(base, _CHUNK)], batch_v)
    pltpu.sync_copy(px_hbm.at[pl.ds(base, _CHUNK)], px_v)
    pltpu.sync_copy(py_hbm.at[pl.ds(base, _CHUNK)], py_v)
    pltpu.sync_copy(pz_hbm.at[pl.ds(base, _CHUNK)], pz_v)

    copies = []
    for j in range(_CHUNK // 128):
        sl = pl.ds(j * 128, 128)
        copies.append(
            pltpu.async_copy(mt_hbm.at[atno_v.at[sl]], m_v.at[sl], sem))
    for c in copies:
        c.wait()

    for q in range(4):
        for mol in range(_N_MOL):
            acc_v[q, mol, :] = jnp.zeros((_LANES,), jnp.float32)

    bmin = batch_v[pl.ds(0, _LANES)][0]
    bmax = batch_v[pl.ds(_CHUNK - _LANES, _LANES)][_LANES - 1]

    def mol_body(mol, _):
        def vec_body(j, carry):
            ax, ay, az, am = carry
            sl = pl.ds(j * _LANES, _LANES)
            keep = batch_v[sl] == mol
            m = jnp.where(keep, m_v[sl], 0.0)
            ax = ax + m * px_v[sl]
            ay = ay + m * py_v[sl]
            az = az + m * pz_v[sl]
            am = am + m
            return ax, ay, az, am

        z = jnp.zeros((_LANES,), jnp.float32)
        ax, ay, az, am = lax.fori_loop(0, _CHUNK // _LANES, vec_body,
                                       (z, z, z, z))
        acc_v[0, mol, :] = ax
        acc_v[1, mol, :] = ay
        acc_v[2, mol, :] = az
        acc_v[3, mol, :] = am
        return 0

    lax.fori_loop(bmin, bmax + 1, mol_body, 0)

    pltpu.sync_copy(acc_v, out_hbm.at[wid])


def _mass_side(px, py, pz, batch1d, atno1d, mt1d):
    mesh = plsc.VectorSubcoreMesh(core_axis_name="c", subcore_axis_name="s")
    k = functools.partial(
        pl.kernel,
        out_type=jax.ShapeDtypeStruct((_NW, 4, _N_MOL, _LANES),
                                      jnp.float32),
        mesh=mesh,
        scratch_types=[
            pltpu.VMEM((_CHUNK,), jnp.float32),
            pltpu.VMEM((_CHUNK,), jnp.float32),
            pltpu.VMEM((_CHUNK,), jnp.float32),
            pltpu.VMEM((_CHUNK,), jnp.int32),
            pltpu.VMEM((_CHUNK,), jnp.int32),
            pltpu.VMEM((_CHUNK,), jnp.float32),
            pltpu.VMEM((4, _N_MOL, _LANES), jnp.float32),
            pltpu.SemaphoreType.DMA,
        ],
    )(_mass_side_kernel)
    return k(px, py, pz, batch1d, atno1d, mt1d)


def _s_side_kernel(x_ref, posT_ref, batch_ref, W1_ref, b1_ref, W2_ref,
                   b2_ref, out_ref):
    i = pl.program_id(0)

    @pl.when(i == 0)
    def _init():
        out_ref[...] = jnp.zeros_like(out_ref)

    x = x_ref[...]
    pT = posT_ref[...]
    b = batch_ref[...]

    px = pT[0:1, :]
    py = pT[1:2, :]
    pz = pT[2:3, :]
    r2 = px * px + py * py + pz * pz

    seg = jnp.where(
        lax.broadcasted_iota(jnp.int32, (_N_MOL, _TILE), 0) == b,
        1.0, 0.0)

    h = jnp.dot(x, W1_ref[...], preferred_element_type=jnp.float32)
    h = h + b1_ref[...]
    h = h * jax.nn.sigmoid(h)
    s = jnp.dot(h, W2_ref[...], preferred_element_type=jnp.float32)
    s = s + b2_ref[...]

    SF = jnp.concatenate([seg * r2, seg * px, seg * py, seg * pz, seg],
                         axis=0)
    out_ref[...] += lax.dot_general(
        SF, s, (((1,), (0,)), ((), ())),
        preferred_element_type=jnp.float32)


def _s_side(x_scalar, posT, batch2, W1, b1r, W2, b2r):
    return pl.pallas_call(
        _s_side_kernel,
        grid=(_GRID,),
        in_specs=[
            pl.BlockSpec((_TILE, _NODE_DIM), lambda i: (i, 0)),
            pl.BlockSpec((3, _TILE), lambda i: (0, i)),
            pl.BlockSpec((1, _TILE), lambda i: (0, i)),
            pl.BlockSpec((_NODE_DIM, _HIDDEN_DIM), lambda i: (0, 0)),
            pl.BlockSpec((1, _HIDDEN_DIM), lambda i: (0, 0)),
            pl.BlockSpec((_HIDDEN_DIM, 1), lambda i: (0, 0)),
            pl.BlockSpec((1, 1), lambda i: (0, 0)),
        ],
        out_specs=pl.BlockSpec((80, 1), lambda i: (0, 0)),
        out_shape=jax.ShapeDtypeStruct((80, 1), jnp.float32),
        compiler_params=pltpu.CompilerParams(
            dimension_semantics=("arbitrary",)),
    )(x_scalar, posT, batch2, W1, b1r, W2, b2r)


def kernel(x_scalar, x_spherical, pos, batch, at_no, masses_table, W1, b1,
           W2, b2):
    del x_spherical
    posT = pos.T
    batch1d = batch.astype(jnp.int32)
    batch2 = batch1d.reshape(1, _N_ATOMS)
    atno1d = at_no.astype(jnp.int32)
    mt1d = jnp.zeros((128,), jnp.float32).at[:_N_ELEM].set(masses_table)
    b1r = b1.reshape(1, _HIDDEN_DIM)
    b2r = b2.reshape(1, 1)

    sc_part = _mass_side(posT[0], posT[1], posT[2], batch1d, atno1d,
                         mt1d)
    accs = _s_side(x_scalar, posT, batch2, W1, b1r, W2, b2r)

    mass = jnp.sum(sc_part, axis=(0, 3))
    A = accs[0:16, 0]
    Bx = accs[16:32, 0]
    By = accs[32:48, 0]
    Bz = accs[48:64, 0]
    C = accs[64:80, 0]
    S = mass[3]
    den = jnp.where(S > 0.0, S, 1.0)
    cx = mass[0] / den
    cy = mass[1] / den
    cz = mass[2] / den
    res = (A - 2.0 * (Bx * cx + By * cy + Bz * cz)
           + C * (cx * cx + cy * cy + cz * cz))
    return res.reshape(_N_MOL, 1)


# speedup vs baseline: 5.2704x; 5.1810x over previous
import functools

import jax
import jax.numpy as jnp
from jax import lax
from jax.experimental import pallas as pl
from jax.experimental.pallas import tpu as pltpu
from jax.experimental.pallas import tpu_sc as plsc

_N_ATOMS = 32768
_N_MOL = 16
_NODE_DIM = 128
_HIDDEN_DIM = 64
_N_ELEM = 119
_TILE = 8192
_GRID = _N_ATOMS // _TILE

_NC = 2
_NS = 16
_NW = _NC * _NS
_CHUNK = _N_ATOMS // _NW
_LANES = 16


def _mass_side_kernel(px_hbm, py_hbm, pz_hbm, batch_hbm, atno_hbm, mt_hbm,
                      out_hbm, px_v, py_v, pz_v, batch_v, atno_v, m_v,
                      mt_v, acc_v, sem):
    cid = lax.axis_index("c")
    sid = lax.axis_index("s")
    wid = cid * _NS + sid
    base = wid * _CHUNK

    sl_in = pl.ds(base, _CHUNK)
    copies = [
        pltpu.async_copy(atno_hbm.at[sl_in], atno_v, sem),
        pltpu.async_copy(batch_hbm.at[sl_in], batch_v, sem),
        pltpu.async_copy(px_hbm.at[sl_in], px_v, sem),
        pltpu.async_copy(py_hbm.at[sl_in], py_v, sem),
        pltpu.async_copy(pz_hbm.at[sl_in], pz_v, sem),
        pltpu.async_copy(mt_hbm, mt_v, sem),
    ]
    for c in copies:
        c.wait()

    tab = [mt_v[pl.ds(t * _LANES, _LANES)] for t in range(8)]

    def gather_body(j, _):
        sl = pl.ds(j * _LANES, _LANES)
        a = atno_v[sl]
        hi = lax.shift_right_logical(a, 4)
        lo = lax.bitwise_and(a, 15)
        m = jnp.zeros((_LANES,), jnp.float32)
        for t in range(8):
            m = jnp.where(hi == t,
                          tab[t].at[lo].get(mode="promise_in_bounds"), m)
        m_v[sl] = m
        return 0

    lax.fori_loop(0, _CHUNK // _LANES, gather_body, 0)

    for q in range(4):
        for mol in range(_N_MOL):
            acc_v[q, mol, :] = jnp.zeros((_LANES,), jnp.float32)

    bmin = batch_v[pl.ds(0, _LANES)][0]
    bmax = batch_v[pl.ds(_CHUNK - _LANES, _LANES)][_LANES - 1]

    def mol_body(mol, _):
        def vec_body(j, carry):
            ax, ay, az, am = carry
            sl = pl.ds(j * _LANES, _LANES)
            keep = batch_v[sl] == mol
            m = jnp.where(keep, m_v[sl], 0.0)
            ax = ax + m * px_v[sl]
            ay = ay + m * py_v[sl]
            az = az + m * pz_v[sl]
            am = am + m
            return ax, ay, az, am

        z = jnp.zeros((_LANES,), jnp.float32)
        ax, ay, az, am = lax.fori_loop(0, _CHUNK // _LANES, vec_body,
                                       (z, z, z, z))
        acc_v[0, mol, :] = ax
        acc_v[1, mol, :] = ay
        acc_v[2, mol, :] = az
        acc_v[3, mol, :] = am
        return 0

    lax.fori_loop(bmin, bmax + 1, mol_body, 0)

    pltpu.sync_copy(acc_v, out_hbm.at[wid])


def _mass_side(px, py, pz, batch1d, atno1d, mt1d):
    mesh = plsc.VectorSubcoreMesh(core_axis_name="c", subcore_axis_name="s")
    k = functools.partial(
        pl.kernel,
        out_type=jax.ShapeDtypeStruct((_NW, 4, _N_MOL, _LANES),
                                      jnp.float32),
        mesh=mesh,
        scratch_types=[
            pltpu.VMEM((_CHUNK,), jnp.float32),
            pltpu.VMEM((_CHUNK,), jnp.float32),
            pltpu.VMEM((_CHUNK,), jnp.float32),
            pltpu.VMEM((_CHUNK,), jnp.int32),
            pltpu.VMEM((_CHUNK,), jnp.int32),
            pltpu.VMEM((_CHUNK,), jnp.float32),
            pltpu.VMEM((128,), jnp.float32),
            pltpu.VMEM((4, _N_MOL, _LANES), jnp.float32),
            pltpu.SemaphoreType.DMA,
        ],
    )(_mass_side_kernel)
    return k(px, py, pz, batch1d, atno1d, mt1d)


def _s_side_kernel(x_ref, posT_ref, batch_ref, W1_ref, b1_ref, W2_ref,
                   b2_ref, out_ref):
    i = pl.program_id(0)

    @pl.when(i == 0)
    def _init():
        out_ref[...] = jnp.zeros_like(out_ref)

    x = x_ref[...]
    pT = posT_ref[...]
    b = batch_ref[...]

    px = pT[0:1, :]
    py = pT[1:2, :]
    pz = pT[2:3, :]
    r2 = px * px + py * py + pz * pz

    seg = jnp.where(
        lax.broadcasted_iota(jnp.int32, (_N_MOL, _TILE), 0) == b,
        1.0, 0.0)

    h = jnp.dot(x, W1_ref[...], preferred_element_type=jnp.float32)
    h = h + b1_ref[...]
    h = h * jax.nn.sigmoid(h)
    s = jnp.dot(h, W2_ref[...], preferred_element_type=jnp.float32)
    s = s + b2_ref[...]

    SF = jnp.concatenate([seg * r2, seg * px, seg * py, seg * pz, seg],
                         axis=0)
    out_ref[...] += lax.dot_general(
        SF, s, (((1,), (0,)), ((), ())),
        preferred_element_type=jnp.float32)


def _s_side(x_scalar, posT, batch2, W1, b1r, W2, b2r):
    return pl.pallas_call(
        _s_side_kernel,
        grid=(_GRID,),
        in_specs=[
            pl.BlockSpec((_TILE, _NODE_DIM), lambda i: (i, 0)),
            pl.BlockSpec((3, _TILE), lambda i: (0, i)),
            pl.BlockSpec((1, _TILE), lambda i: (0, i)),
            pl.BlockSpec((_NODE_DIM, _HIDDEN_DIM), lambda i: (0, 0)),
            pl.BlockSpec((1, _HIDDEN_DIM), lambda i: (0, 0)),
            pl.BlockSpec((_HIDDEN_DIM, 1), lambda i: (0, 0)),
            pl.BlockSpec((1, 1), lambda i: (0, 0)),
        ],
        out_specs=pl.BlockSpec((80, 1), lambda i: (0, 0)),
        out_shape=jax.ShapeDtypeStruct((80, 1), jnp.float32),
        compiler_params=pltpu.CompilerParams(
            dimension_semantics=("arbitrary",)),
    )(x_scalar, posT, batch2, W1, b1r, W2, b2r)


def kernel(x_scalar, x_spherical, pos, batch, at_no, masses_table, W1, b1,
           W2, b2):
    del x_spherical
    posT = pos.T
    batch1d = batch.astype(jnp.int32)
    batch2 = batch1d.reshape(1, _N_ATOMS)
    atno1d = at_no.astype(jnp.int32)
    mt1d = jnp.zeros((128,), jnp.float32).at[:_N_ELEM].set(masses_table)
    b1r = b1.reshape(1, _HIDDEN_DIM)
    b2r = b2.reshape(1, 1)

    sc_part = _mass_side(posT[0], posT[1], posT[2], batch1d, atno1d,
                         mt1d)
    accs = _s_side(x_scalar, posT, batch2, W1, b1r, W2, b2r)

    mass = jnp.sum(sc_part, axis=(0, 3))
    A = accs[0:16, 0]
    Bx = accs[16:32, 0]
    By = accs[32:48, 0]
    Bz = accs[48:64, 0]
    C = accs[64:80, 0]
    S = mass[3]
    den = jnp.where(S > 0.0, S, 1.0)
    cx = mass[0] / den
    cy = mass[1] / den
    cz = mass[2] / den
    res = (A - 2.0 * (Bx * cx + By * cy + Bz * cz)
           + C * (cx * cx + cy * cy + cz * cz))
    return res.reshape(_N_MOL, 1)
